# baseline (speedup 1.0000x reference)
import functools

import jax
import jax.numpy as jnp
from jax.experimental import pallas as pl
from jax.experimental.pallas import tpu as pltpu


def _transform_kernel(xT_ref, U0_ref, b0_ref, out_ref):
    out_ref[...] = jax.lax.dot_general(
        U0_ref[...], xT_ref[...], (((0,), (0,)), ((), ())),
        preferred_element_type=jnp.float32) + b0_ref[...]


def _transform(xT, U0, b0c):
    D, N = xT.shape
    bn = 512
    return pl.pallas_call(
        _transform_kernel,
        grid=(N // bn,),
        in_specs=[
            pl.BlockSpec((D, bn), lambda j: (0, j)),
            pl.BlockSpec((D, D), lambda j: (0, 0)),
            pl.BlockSpec((D, 1), lambda j: (0, 0)),
        ],
        out_specs=pl.BlockSpec((D, bn), lambda j: (0, j)),
        out_shape=jax.ShapeDtypeStruct((D, N), jnp.float32),
    )(xT, U0, b0c)


def _layer_kernel(nk, y_ref, A_ref, U_ref, out_ref, acc_ref, deg_ref):
    k = pl.program_id(1)

    @pl.when(k == 0)
    def _init():
        acc_ref[...] = jnp.zeros_like(acc_ref)
        deg_ref[...] = jnp.zeros_like(deg_ref)

    A_raw = A_ref[...]
    Af = A_raw.astype(jnp.bfloat16)
    yb = y_ref[...].astype(jnp.bfloat16)
    acc_ref[...] += jnp.dot(yb, Af, preferred_element_type=jnp.float32)
    deg_ref[...] += jnp.sum(A_raw, axis=0, keepdims=True).astype(jnp.float32)

    @pl.when(k == nk - 1)
    def _epilogue():
        agg = acc_ref[...] / deg_ref[...]
        out_ref[...] = jnp.maximum(
            jnp.dot(U_ref[...], agg, preferred_element_type=jnp.float32), 0.0)


def _layer(y, adj, U, bi=512, bk=1024):
    D, N = y.shape
    ni, nk = N // bi, N // bk
    return pl.pallas_call(
        functools.partial(_layer_kernel, nk),
        grid=(ni, nk),
        in_specs=[
            pl.BlockSpec((D, bk), lambda i, k: (0, k)),
            pl.BlockSpec((bk, bi), lambda i, k: (k, i)),
            pl.BlockSpec((D, D), lambda i, k: (0, 0)),
        ],
        out_specs=pl.BlockSpec((D, bi), lambda i, k: (0, i)),
        out_shape=jax.ShapeDtypeStruct((D, N), jnp.float32),
        scratch_shapes=[
            pltpu.VMEM((D, bi), jnp.float32),
            pltpu.VMEM((1, bi), jnp.float32),
        ],
        compiler_params=pltpu.CompilerParams(
            dimension_semantics=("parallel", "arbitrary")),
    )(y, adj, U)


def kernel(x, adj_mat, U0, b0, U1, U2):
    N, D = x.shape
    xT = x.T
    y0 = _transform(xT, U0, b0.reshape(D, 1))
    y1 = _layer(y0, adj_mat, U1)
    y2 = _layer(y1, adj_mat, U2)
    return y2.T


# speedup vs baseline: 1.4790x; 1.4790x over previous
import functools

import jax
import jax.numpy as jnp
from jax.experimental import pallas as pl
from jax.experimental.pallas import tpu as pltpu


def _transform_kernel(xT_ref, U0_ref, b0_ref, out_ref):
    out_ref[...] = jax.lax.dot_general(
        U0_ref[...], xT_ref[...], (((0,), (0,)), ((), ())),
        preferred_element_type=jnp.float32) + b0_ref[...]


def _transform(xT, U0, b0c):
    D, N = xT.shape
    bn = 512
    return pl.pallas_call(
        _transform_kernel,
        grid=(N // bn,),
        in_specs=[
            pl.BlockSpec((D, bn), lambda j: (0, j)),
            pl.BlockSpec((D, D), lambda j: (0, 0)),
            pl.BlockSpec((D, 1), lambda j: (0, 0)),
        ],
        out_specs=pl.BlockSpec((D, bn), lambda j: (0, j)),
        out_shape=jax.ShapeDtypeStruct((D, N), jnp.float32),
    )(xT, U0, b0c)


def _layer_kernel(nk, y_ref, A_ref, U_ref, out_ref, acc_ref, deg_ref):
    k = pl.program_id(1)

    @pl.when(k == 0)
    def _init():
        acc_ref[...] = jnp.zeros_like(acc_ref)
        deg_ref[...] = jnp.zeros_like(deg_ref)

    A_raw = A_ref[...]
    Af = A_raw.astype(jnp.bfloat16)
    yb = y_ref[...].astype(jnp.bfloat16)
    acc_ref[...] += jnp.dot(yb, Af, preferred_element_type=jnp.float32)
    deg_ref[...] += jnp.sum(A_raw, axis=0, keepdims=True).astype(jnp.float32)

    @pl.when(k == nk - 1)
    def _epilogue():
        agg = acc_ref[...] * (1.0 / deg_ref[...])
        out_ref[...] = jnp.maximum(
            jnp.dot(U_ref[...], agg, preferred_element_type=jnp.float32), 0.0)


def _layer(y, adj, U, bi=4096, bk=512):
    D, N = y.shape
    ni, nk = N // bi, N // bk
    return pl.pallas_call(
        functools.partial(_layer_kernel, nk),
        grid=(ni, nk),
        in_specs=[
            pl.BlockSpec((D, bk), lambda i, k: (0, k)),
            pl.BlockSpec((bk, bi), lambda i, k: (k, i)),
            pl.BlockSpec((D, D), lambda i, k: (0, 0)),
        ],
        out_specs=pl.BlockSpec((D, bi), lambda i, k: (0, i)),
        out_shape=jax.ShapeDtypeStruct((D, N), jnp.float32),
        scratch_shapes=[
            pltpu.VMEM((D, bi), jnp.float32),
            pltpu.VMEM((1, bi), jnp.float32),
        ],
        compiler_params=pltpu.CompilerParams(
            dimension_semantics=("parallel", "arbitrary")),
    )(y, adj, U)


def kernel(x, adj_mat, U0, b0, U1, U2):
    N, D = x.shape
    xT = x.T
    y0 = _transform(xT, U0, b0.reshape(D, 1))
    y1 = _layer(y0, adj_mat, U1)
    y2 = _layer(y1, adj_mat, U2)
    return y2.T


# speedup vs baseline: 2.2213x; 1.5020x over previous
import functools

import jax
import jax.numpy as jnp
from jax.experimental import pallas as pl
from jax.experimental.pallas import tpu as pltpu


def _fused_kernel(nk, xT_ref, A_ref, U0_ref, b0_ref, Us_ref, out_ref,
                  acc_ref, deg_ref, a8_ref, y1_ref):
    l = pl.program_id(0)
    k = pl.program_id(1)

    @pl.when(k == 0)
    def _reset_acc():
        acc_ref[...] = jnp.zeros_like(acc_ref)

    @pl.when(jnp.logical_and(l == 0, k == 0))
    def _reset_deg():
        deg_ref[...] = jnp.zeros_like(deg_ref)

    @pl.when(l == 0)
    def _layer1_step():
        A_raw = A_ref[...]
        Af = A_raw.astype(jnp.bfloat16)
        a8_ref[k] = A_raw.astype(jnp.int8)
        y0 = jax.lax.dot_general(
            U0_ref[...], xT_ref[...], (((0,), (0,)), ((), ())),
            preferred_element_type=jnp.float32) + b0_ref[...]
        acc_ref[...] += jnp.dot(y0.astype(jnp.bfloat16), Af,
                                preferred_element_type=jnp.float32)
        deg_ref[...] += jnp.sum(A_raw, axis=0, keepdims=True).astype(jnp.float32)

        @pl.when(k == nk - 1)
        def _layer1_out():
            agg = acc_ref[...] * (1.0 / deg_ref[...])
            y1_ref[...] = jnp.maximum(
                jnp.dot(Us_ref[0], agg, preferred_element_type=jnp.float32), 0.0)

    @pl.when(l == 1)
    def _layer2_step():
        bk = a8_ref.shape[1]
        Af = a8_ref[k].astype(jnp.bfloat16)
        yb = y1_ref[:, pl.ds(k * bk, bk)].astype(jnp.bfloat16)
        acc_ref[...] += jnp.dot(yb, Af, preferred_element_type=jnp.float32)

        @pl.when(k == nk - 1)
        def _layer2_out():
            agg = acc_ref[...] * (1.0 / deg_ref[...])
            out_ref[...] = jnp.maximum(
                jnp.dot(Us_ref[1], agg, preferred_element_type=jnp.float32), 0.0)


def kernel(x, adj_mat, U0, b0, U1, U2):
    N, D = x.shape
    bk = 512
    nk = N // bk
    xT = x.T
    Us = jnp.stack([U1, U2])
    b0c = b0.reshape(D, 1)
    yT = pl.pallas_call(
        functools.partial(_fused_kernel, nk),
        grid=(2, nk),
        in_specs=[
            pl.BlockSpec((D, bk),
                         lambda l, k: (0, jnp.where(l == 0, k, nk - 1))),
            pl.BlockSpec((bk, N),
                         lambda l, k: (jnp.where(l == 0, k, nk - 1), 0)),
            pl.BlockSpec((D, D), lambda l, k: (0, 0)),
            pl.BlockSpec((D, 1), lambda l, k: (0, 0)),
            pl.BlockSpec((2, D, D), lambda l, k: (0, 0, 0)),
        ],
        out_specs=pl.BlockSpec((D, N), lambda l, k: (0, 0)),
        out_shape=jax.ShapeDtypeStruct((D, N), jnp.float32),
        scratch_shapes=[
            pltpu.VMEM((D, N), jnp.float32),
            pltpu.VMEM((1, N), jnp.float32),
            pltpu.VMEM((nk, bk, N), jnp.int8),
            pltpu.VMEM((D, N), jnp.float32),
        ],
        compiler_params=pltpu.CompilerParams(
            dimension_semantics=("arbitrary", "arbitrary")),
    )(xT, adj_mat, U0, b0c, Us)
    return yT.T
